# Initial kernel scaffold; baseline (speedup 1.0000x reference)
import jax
import jax.numpy as jnp
from jax.experimental import pallas as pl


def kernel(x):
    raise NotImplementedError("write your pallas kernel here")


# speedup vs baseline: 1.0067x; 1.0067x over previous
import functools

import jax
import jax.numpy as jnp
from jax import lax
from jax.experimental import pallas as pl
from jax.experimental.pallas import tpu as pltpu
from jax.experimental.pallas import tpu_sc as plsc

_BINS = 2048
_NC, _NS, _L = 2, 16, 16
_NW = _NC * _NS
_N = 33554432
_PER_W = _N // _NW
_CHUNK = 32768
_NPAIR = _PER_W // (2 * _CHUNK)

_mesh = plsc.VectorSubcoreMesh(core_axis_name="c", subcore_axis_name="s")


@functools.partial(
    pl.kernel,
    mesh=_mesh,
    compiler_params=pltpu.CompilerParams(needs_layout_passes=False),
    out_type=jax.ShapeDtypeStruct((2 * _NW * _L,), jnp.float32),
    scratch_types=[
        pltpu.VMEM((_CHUNK,), jnp.float32),
        pltpu.VMEM((_CHUNK,), jnp.float32),
        pltpu.VMEM((2 * _L,), jnp.float32),
        pltpu.SemaphoreType.DMA,
        pltpu.SemaphoreType.DMA,
    ],
)
def _minmax_k(x_hbm, out_hbm, buf0, buf1, res_v, sem0, sem1):
    wid = lax.axis_index("s") * _NC + lax.axis_index("c")
    base = wid * _PER_W

    def scan_buf(buf, carry):
        def body(i, c):
            mn, mx = c
            v = buf[pl.ds(i * _L, _L)]
            return jnp.minimum(mn, v), jnp.maximum(mx, v)
        return lax.fori_loop(0, _CHUNK // _L, body, carry)

    pltpu.async_copy(x_hbm.at[pl.ds(base, _CHUNK)], buf0, sem0)

    def outer(j, carry):
        b = base + 2 * j * _CHUNK
        pltpu.async_copy(x_hbm.at[pl.ds(b + _CHUNK, _CHUNK)], buf1, sem1)
        pltpu.make_async_copy(x_hbm.at[pl.ds(b, _CHUNK)], buf0, sem0).wait()
        carry = scan_buf(buf0, carry)

        @pl.when(j < _NPAIR - 1)
        def _():
            pltpu.async_copy(
                x_hbm.at[pl.ds(b + 2 * _CHUNK, _CHUNK)], buf0, sem0)

        pltpu.make_async_copy(
            x_hbm.at[pl.ds(b + _CHUNK, _CHUNK)], buf1, sem1).wait()
        carry = scan_buf(buf1, carry)
        return carry

    init = (jnp.full((_L,), jnp.inf, jnp.float32),
            jnp.full((_L,), -jnp.inf, jnp.float32))
    mn, mx = lax.fori_loop(0, _NPAIR, outer, init)
    res_v[pl.ds(0, _L)] = mn
    res_v[pl.ds(_L, _L)] = mx
    pltpu.sync_copy(res_v.at[pl.ds(0, _L)], out_hbm.at[pl.ds(wid * _L, _L)])
    pltpu.sync_copy(res_v.at[pl.ds(_L, _L)],
                    out_hbm.at[pl.ds((_NW + wid) * _L, _L)])


@functools.partial(
    pl.kernel,
    mesh=_mesh,
    compiler_params=pltpu.CompilerParams(needs_layout_passes=False),
    out_type=jax.ShapeDtypeStruct((_NW, _BINS), jnp.float32),
    scratch_types=[
        pltpu.VMEM((_CHUNK,), jnp.float32),
        pltpu.VMEM((_CHUNK,), jnp.float32),
        pltpu.VMEM((_L * _BINS,), jnp.float32),
        pltpu.VMEM((_BINS,), jnp.float32),
        pltpu.VMEM((2 * _L,), jnp.float32),
        pltpu.SemaphoreType.DMA,
        pltpu.SemaphoreType.DMA,
    ],
)
def _hist_k(x_hbm, mw_hbm, out_hbm, buf0, buf1, hist16, histl, mw_v,
            sem0, sem1):
    wid = lax.axis_index("s") * _NC + lax.axis_index("c")
    base = wid * _PER_W

    pltpu.sync_copy(mw_hbm, mw_v)
    mnv = mw_v[pl.ds(0, _L)]
    wv = mw_v[pl.ds(_L, _L)]
    lane_off = lax.iota(jnp.int32, _L) * _BINS
    ones = jnp.full((_L,), 1.0, jnp.float32)
    zeros = jnp.zeros((_L,), jnp.float32)

    def zero_body(i, _):
        hist16[pl.ds(i * _L, _L)] = zeros
        return 0
    lax.fori_loop(0, _L * _BINS // _L, zero_body, 0)

    def scan_buf(buf, carry):
        def body(i, c):
            v = buf[pl.ds(i * _L, _L)]
            t = (v - mnv) / wv
            idx = jnp.minimum(t.astype(jnp.int32), _BINS - 1)
            plsc.addupdate_scatter(hist16, [idx + lane_off], ones)
            return c
        return lax.fori_loop(0, _CHUNK // _L, body, carry)

    pltpu.async_copy(x_hbm.at[pl.ds(base, _CHUNK)], buf0, sem0)

    def outer(j, carry):
        b = base + 2 * j * _CHUNK
        pltpu.async_copy(x_hbm.at[pl.ds(b + _CHUNK, _CHUNK)], buf1, sem1)
        pltpu.make_async_copy(x_hbm.at[pl.ds(b, _CHUNK)], buf0, sem0).wait()
        carry = scan_buf(buf0, carry)

        @pl.when(j < _NPAIR - 1)
        def _():
            pltpu.async_copy(
                x_hbm.at[pl.ds(b + 2 * _CHUNK, _CHUNK)], buf0, sem0)

        pltpu.make_async_copy(
            x_hbm.at[pl.ds(b + _CHUNK, _CHUNK)], buf1, sem1).wait()
        carry = scan_buf(buf1, carry)
        return carry

    lax.fori_loop(0, _NPAIR, outer, 0)

    def red_outer(bblk, _):
        def red_inner(l, acc):
            return acc + hist16[pl.ds(l * _BINS + bblk * _L, _L)]
        acc = lax.fori_loop(0, _L, red_inner, zeros)
        histl[pl.ds(bblk * _L, _L)] = acc
        return 0
    lax.fori_loop(0, _BINS // _L, red_outer, 0)
    pltpu.sync_copy(histl, out_hbm.at[wid])


def kernel(x):
    part = _minmax_k(x)
    mn = jnp.min(part[: _NW * _L])
    mx = jnp.max(part[_NW * _L:])
    bw = (mx - mn) / _BINS
    safe_w = jnp.where(bw == 0, jnp.float32(1.0), bw)
    mw = jnp.concatenate([
        jnp.full((_L,), 1.0, jnp.float32) * mn,
        jnp.full((_L,), 1.0, jnp.float32) * safe_w,
    ])
    parts = _hist_k(x, mw)
    hist = jnp.sum(parts, axis=0)
    return jnp.concatenate([jnp.stack([mn, mx]), hist])
